# Initial kernel scaffold; baseline (speedup 1.0000x reference)
import functools

import jax
import jax.numpy as jnp
from jax import lax
from jax.experimental import pallas as pl
from jax.experimental.pallas import tpu as pltpu
from jax.experimental.pallas import tpu_sc as plsc

NUM_CORES = 2
NUM_SUBCORES = 16
CH = 80
DC = 125
LANES = 16


def _make_spmm(n_nodes, feat, n_chunks_total, alpha, beta):
    assert n_nodes % (NUM_SUBCORES * DC) == 0
    assert n_chunks_total % NUM_SUBCORES == 0
    npc = n_chunks_total // NUM_SUBCORES
    rpt = n_nodes // NUM_SUBCORES
    n_drain = rpt // DC

    mesh = plsc.VectorSubcoreMesh(
        core_axis_name="c", subcore_axis_name="s",
        num_cores=NUM_CORES, num_subcores=NUM_SUBCORES)

    @functools.partial(
        pl.kernel,
        out_type=jax.ShapeDtypeStruct((NUM_CORES * n_nodes, feat), jnp.float32),
        mesh=mesh,
        scratch_types=[
            pltpu.VMEM_SHARED((n_nodes, feat), jnp.float32),
            pltpu.VMEM((npc, CH), jnp.int32),
            pltpu.VMEM((npc, CH), jnp.int32),
            pltpu.VMEM((npc, CH), jnp.float32),
            pltpu.VMEM((CH, feat), jnp.float32),
            pltpu.VMEM((DC, feat), jnp.float32),
            pltpu.VMEM((DC, feat), jnp.float32),
            pltpu.SemaphoreType.DMA,
        ],
    )
    def spmm(t_ref, c_ref, src_ref, dst_ref, val_ref, out_ref,
             acc, src_v, dst_v, val_v, rows_v, drain_v, c_v, sem):
        cid = lax.axis_index("c")
        sid = lax.axis_index("s")
        cbase = sid * npc
        rbase = sid * rpt

        pltpu.sync_copy(src_ref.at[pl.ds(cbase, npc)], src_v)
        pltpu.sync_copy(dst_ref.at[pl.ds(cbase, npc)], dst_v)
        pltpu.sync_copy(val_ref.at[pl.ds(cbase, npc)], val_v)

        off = cid * n_nodes

        def shift_row(i, _):
            for g in range(CH // LANES):
                sl = pl.ds(g * LANES, LANES)
                src_v[i, sl] = src_v[i, sl] + off
            return 0
        lax.fori_loop(0, npc, shift_row, 0)

        zeros = jnp.zeros((LANES,), jnp.float32)

        def zero_row(r, _):
            for j in range(feat // LANES):
                drain_v[r, pl.ds(j * LANES, LANES)] = zeros
            return 0
        lax.fori_loop(0, DC, zero_row, 0)
        for k in range(n_drain):
            pltpu.sync_copy(drain_v, acc.at[pl.ds(rbase + k * DC, DC)])
        plsc.subcore_barrier()

        def chunk_body(i, _):
            pltpu.async_copy(t_ref.at[src_v.at[i]], rows_v, sem).wait()

            def edge_body(e, _):
                v = val_v[i, e]
                for j in range(feat // LANES):
                    sl = pl.ds(j * LANES, LANES)
                    rows_v[e, sl] = rows_v[e, sl] * v
                return 0
            lax.fori_loop(0, CH, edge_body, 0)
            pltpu.sync_copy(rows_v, acc.at[dst_v.at[i]], add=True)
            return 0
        lax.fori_loop(0, npc, chunk_body, 0)
        plsc.subcore_barrier()

        obase = cid * n_nodes + rbase
        for k in range(n_drain):
            pltpu.sync_copy(acc.at[pl.ds(rbase + k * DC, DC)], drain_v)
            if beta != 0.0:
                pltpu.sync_copy(c_ref.at[pl.ds(obase + k * DC, DC)], c_v)

                def comb_row(r, _):
                    for j in range(feat // LANES):
                        sl = pl.ds(j * LANES, LANES)
                        drain_v[r, sl] = drain_v[r, sl] * alpha + c_v[r, sl] * beta
                    return 0
                lax.fori_loop(0, DC, comb_row, 0)
            pltpu.sync_copy(drain_v, out_ref.at[pl.ds(obase + k * DC, DC)])

    return spmm


def _tc_combine(ts, weight, prelu_a):
    b, n, fin = ts[0].shape
    k_order = weight.shape[0]
    fout = weight.shape[2]
    bn = 1000
    assert n % bn == 0

    def body(*refs):
        t_refs = refs[:k_order]
        w_ref, a_ref, o_ref = refs[k_order:]
        acc = jnp.zeros((bn, fout), jnp.float32)
        for k in range(k_order):
            acc = acc + jnp.dot(t_refs[k][0], w_ref[k],
                                preferred_element_type=jnp.float32)
        a = a_ref[0]
        o_ref[0] = jnp.where(acc >= 0, acc, a * acc)

    t_spec = pl.BlockSpec((1, bn, fin), lambda i, j: (i, j, 0))
    return pl.pallas_call(
        body,
        grid=(b, n // bn),
        in_specs=[t_spec] * k_order + [
            pl.BlockSpec((k_order, fin, fout), lambda i, j: (0, 0, 0)),
            pl.BlockSpec(memory_space=pltpu.SMEM),
        ],
        out_specs=pl.BlockSpec((1, bn, fout), lambda i, j: (i, j, 0)),
        out_shape=jax.ShapeDtypeStruct((b, n, fout), jnp.float32),
    )(*ts, weight, prelu_a)


def kernel(x, lap_indices, lap_values, weight, prelu_a):
    b, n, fin = x.shape
    k_order = weight.shape[0]
    e = lap_values.shape[0]

    src = lap_indices[0]
    dst = lap_indices[1]
    vals = lap_values
    pad = (-e) % (NUM_SUBCORES * CH)
    if pad:
        src = jnp.concatenate([src, jnp.zeros((pad,), jnp.int32)])
        dst = jnp.concatenate([dst, jnp.zeros((pad,), jnp.int32)])
        vals = jnp.concatenate([vals, jnp.zeros((pad,), jnp.float32)])
    n_chunks_total = (e + pad) // CH
    src2d = src.reshape(n_chunks_total, CH)
    dst2d = dst.reshape(n_chunks_total, CH)
    val2d = vals.reshape(n_chunks_total, CH)

    xf = x.reshape(b * n, fin)
    spmm_first = _make_spmm(n, fin, n_chunks_total, 1.0, 0.0)
    spmm_rec = _make_spmm(n, fin, n_chunks_total, 2.0, -1.0)

    ts = [xf, spmm_first(xf, xf, src2d, dst2d, val2d)]
    for _ in range(2, k_order):
        ts.append(spmm_rec(ts[-1], ts[-2], src2d, dst2d, val2d))

    ts3d = [t.reshape(b, n, fin) for t in ts]
    return _tc_combine(ts3d, weight, prelu_a)


# speedup vs baseline: 20.0346x; 20.0346x over previous
import functools

import jax
import jax.numpy as jnp
from jax import lax
from jax.experimental import pallas as pl
from jax.experimental.pallas import tpu as pltpu
from jax.experimental.pallas import tpu_sc as plsc

NUM_CORES = 2
NUM_SUBCORES = 16
CH = 80
GC = 16
DC = 80
LANES = 16


def _make_spmm(n_nodes, feat, n_chunks_total, alpha, beta):
    assert n_nodes % (NUM_SUBCORES * DC) == 0
    assert n_chunks_total % (NUM_SUBCORES * GC) == 0
    npc = n_chunks_total // NUM_SUBCORES
    ngr = npc // GC
    rpt = n_nodes // NUM_SUBCORES
    n_drain = rpt // DC

    mesh = plsc.VectorSubcoreMesh(
        core_axis_name="c", subcore_axis_name="s",
        num_cores=NUM_CORES, num_subcores=NUM_SUBCORES)

    @functools.partial(
        pl.kernel,
        out_type=jax.ShapeDtypeStruct((NUM_CORES * n_nodes, feat), jnp.float32),
        mesh=mesh,
        scratch_types=[
            pltpu.VMEM_SHARED((n_nodes, feat), jnp.float32),
            pltpu.VMEM((GC, CH), jnp.float32),
            pltpu.VMEM((CH,), jnp.int32),
            pltpu.VMEM((CH,), jnp.int32),
            pltpu.VMEM((CH, feat), jnp.float32),
            pltpu.VMEM((DC, feat), jnp.float32),
            pltpu.VMEM((DC, feat), jnp.float32),
            pltpu.SemaphoreType.DMA,
        ],
    )
    def spmm(t_ref, c_ref, src_ref, dst_ref, val_ref, out_ref,
             acc, val_v, idx_b, dst_b, rows_v, drain_v, c_v, sem):
        cid = lax.axis_index("c")
        sid = lax.axis_index("s")
        cbase = sid * npc
        rbase = sid * rpt

        off = cid * n_nodes

        zeros = jnp.zeros((LANES,), jnp.float32)

        def zero_row(r, _):
            for j in range(feat // LANES):
                drain_v[r, pl.ds(j * LANES, LANES)] = zeros
            return 0
        lax.fori_loop(0, DC, zero_row, 0)
        for k in range(n_drain):
            pltpu.sync_copy(drain_v, acc.at[pl.ds(rbase + k * DC, DC)])
        plsc.subcore_barrier()

        def group_body(gi, _):
            gb = cbase + gi * GC
            pltpu.sync_copy(val_ref.at[pl.ds(gb, GC)], val_v)

            def chunk_body(i, _):
                eb = pl.multiple_of((gb + i) * CH, LANES)
                pltpu.sync_copy(src_ref.at[pl.ds(eb, CH)], idx_b)
                pltpu.sync_copy(dst_ref.at[pl.ds(eb, CH)], dst_b)
                for g in range(CH // LANES):
                    sl = pl.ds(g * LANES, LANES)
                    idx_b[sl] = idx_b[sl] + off
                pltpu.async_copy(t_ref.at[idx_b], rows_v, sem).wait()

                def scale_group(g, _):
                    gs = pl.multiple_of(g * LANES, LANES)
                    v16 = val_v[i, pl.ds(gs, LANES)]
                    for ei in range(LANES):
                        v = v16[ei]
                        e = gs + ei
                        for j in range(feat // LANES):
                            sl = pl.ds(j * LANES, LANES)
                            rows_v[e, sl] = rows_v[e, sl] * v
                    return 0
                lax.fori_loop(0, CH // LANES, scale_group, 0)
                pltpu.sync_copy(rows_v, acc.at[dst_b], add=True)
                return 0
            lax.fori_loop(0, GC, chunk_body, 0)
            return 0
        lax.fori_loop(0, ngr, group_body, 0)
        plsc.subcore_barrier()

        obase = cid * n_nodes + rbase
        for k in range(n_drain):
            pltpu.sync_copy(acc.at[pl.ds(rbase + k * DC, DC)], drain_v)
            if beta != 0.0:
                pltpu.sync_copy(c_ref.at[pl.ds(obase + k * DC, DC)], c_v)

                def comb_row(r, _):
                    for j in range(feat // LANES):
                        sl = pl.ds(j * LANES, LANES)
                        drain_v[r, sl] = drain_v[r, sl] * alpha + c_v[r, sl] * beta
                    return 0
                lax.fori_loop(0, DC, comb_row, 0)
            pltpu.sync_copy(drain_v, out_ref.at[pl.ds(obase + k * DC, DC)])

    return spmm


def _tc_combine(ts, n, weight, prelu_a):
    b, _, fin = ts[0].shape
    k_order = weight.shape[0]
    fout = weight.shape[2]
    bn = 1000
    assert n % bn == 0

    def body(*refs):
        t_refs = refs[:k_order]
        w_ref, a_ref, o_ref = refs[k_order:]
        acc = jnp.zeros((bn, fout), jnp.float32)
        for k in range(k_order):
            acc = acc + jnp.dot(t_refs[k][0], w_ref[k],
                                preferred_element_type=jnp.float32)
        a = a_ref[0]
        o_ref[0] = jnp.where(acc >= 0, acc, a * acc)

    t_spec = pl.BlockSpec((1, bn, fin), lambda i, j: (i, j, 0))
    return pl.pallas_call(
        body,
        grid=(b, n // bn),
        in_specs=[t_spec] * k_order + [
            pl.BlockSpec((k_order, fin, fout), lambda i, j: (0, 0, 0)),
            pl.BlockSpec(memory_space=pltpu.SMEM),
        ],
        out_specs=pl.BlockSpec((1, bn, fout), lambda i, j: (i, j, 0)),
        out_shape=jax.ShapeDtypeStruct((b, n, fout), jnp.float32),
    )(*ts, weight, prelu_a)


def kernel(x, lap_indices, lap_values, weight, prelu_a):
    b, n, fin = x.shape
    k_order = weight.shape[0]
    e = lap_values.shape[0]

    src = lap_indices[0]
    dst = lap_indices[1]
    vals = lap_values
    pad = (-e) % (NUM_SUBCORES * CH * GC)
    if pad:
        src = jnp.concatenate([src, jnp.zeros((pad,), jnp.int32)])
        dst = jnp.concatenate([dst, jnp.zeros((pad,), jnp.int32)])
        vals = jnp.concatenate([vals, jnp.zeros((pad,), jnp.float32)])
    n_chunks_total = (e + pad) // CH
    val2d = vals.reshape(n_chunks_total, CH)

    n_pad = ((n + NUM_SUBCORES * DC - 1) // (NUM_SUBCORES * DC)) * (NUM_SUBCORES * DC)
    xp = jnp.pad(x, ((0, 0), (0, n_pad - n), (0, 0)))
    xf = xp.reshape(b * n_pad, fin)
    spmm_first = _make_spmm(n_pad, fin, n_chunks_total, 1.0, 0.0)
    spmm_rec = _make_spmm(n_pad, fin, n_chunks_total, 2.0, -1.0)

    ts = [xf, spmm_first(xf, xf, src, dst, val2d)]
    for _ in range(2, k_order):
        ts.append(spmm_rec(ts[-1], ts[-2], src, dst, val2d))

    ts3d = [t.reshape(b, n_pad, fin) for t in ts]
    return _tc_combine(ts3d, n, weight, prelu_a)


# speedup vs baseline: 22.6325x; 1.1297x over previous
import functools

import jax
import jax.numpy as jnp
from jax import lax
from jax.experimental import pallas as pl
from jax.experimental.pallas import tpu as pltpu
from jax.experimental.pallas import tpu_sc as plsc

NUM_CORES = 2
NUM_SUBCORES = 16
CH = 128
GC = 16
DC = 80
LANES = 16


def _make_spmm(n_nodes, feat, n_chunks_total, alpha, beta):
    assert n_nodes % (NUM_SUBCORES * DC) == 0
    assert n_chunks_total % (NUM_SUBCORES * GC) == 0
    npc = n_chunks_total // NUM_SUBCORES
    ngr = npc // GC
    rpt = n_nodes // NUM_SUBCORES
    n_drain = rpt // DC

    mesh = plsc.VectorSubcoreMesh(
        core_axis_name="c", subcore_axis_name="s",
        num_cores=NUM_CORES, num_subcores=NUM_SUBCORES)

    @functools.partial(
        pl.kernel,
        out_type=jax.ShapeDtypeStruct((NUM_CORES * n_nodes, feat), jnp.float32),
        mesh=mesh,
        scratch_types=[
            pltpu.VMEM_SHARED((n_nodes, feat), jnp.float32),
            pltpu.VMEM((GC, CH), jnp.float32),
            pltpu.VMEM((CH,), jnp.int32),
            pltpu.VMEM((CH,), jnp.int32),
            pltpu.VMEM((CH, feat), jnp.float32),
            pltpu.VMEM((DC, feat), jnp.float32),
            pltpu.VMEM((DC, feat), jnp.float32),
            pltpu.SemaphoreType.DMA,
        ],
    )
    def spmm(t_ref, c_ref, src_ref, dst_ref, val_ref, out_ref,
             acc, val_v, idx_b, dst_b, rows_v, drain_v, c_v, sem):
        cid = lax.axis_index("c")
        sid = lax.axis_index("s")
        cbase = sid * npc
        rbase = sid * rpt

        off = cid * n_nodes

        zeros = jnp.zeros((LANES,), jnp.float32)

        def zero_row(r, _):
            for j in range(feat // LANES):
                drain_v[r, pl.ds(j * LANES, LANES)] = zeros
            return 0
        lax.fori_loop(0, DC, zero_row, 0)
        for k in range(n_drain):
            pltpu.sync_copy(drain_v, acc.at[pl.ds(rbase + k * DC, DC)])
        plsc.subcore_barrier()

        def group_body(gi, _):
            gb = cbase + gi * GC
            pltpu.sync_copy(val_ref.at[pl.ds(gb, GC)], val_v)

            def chunk_body(i, _):
                eb = pl.multiple_of((gb + i) * CH, LANES)
                pltpu.sync_copy(src_ref.at[pl.ds(eb, CH)], idx_b)
                pltpu.sync_copy(dst_ref.at[pl.ds(eb, CH)], dst_b)
                for g in range(CH // LANES):
                    sl = pl.ds(g * LANES, LANES)
                    idx_b[sl] = idx_b[sl] + off
                pltpu.async_copy(t_ref.at[idx_b], rows_v, sem).wait()

                def scale_group(g, _):
                    gs = pl.multiple_of(g * LANES, LANES)
                    v16 = val_v[i, pl.ds(gs, LANES)]
                    for ei in range(LANES):
                        v = v16[ei]
                        e = gs + ei
                        for j in range(feat // LANES):
                            sl = pl.ds(j * LANES, LANES)
                            rows_v[e, sl] = rows_v[e, sl] * v
                    return 0
                lax.fori_loop(0, CH // LANES, scale_group, 0)
                pltpu.sync_copy(rows_v, acc.at[dst_b], add=True)
                return 0
            lax.fori_loop(0, GC, chunk_body, 0)
            return 0
        lax.fori_loop(0, ngr, group_body, 0)
        plsc.subcore_barrier()

        obase = cid * n_nodes + rbase
        for k in range(n_drain):
            pltpu.sync_copy(acc.at[pl.ds(rbase + k * DC, DC)], drain_v)
            if beta != 0.0:
                pltpu.sync_copy(c_ref.at[pl.ds(obase + k * DC, DC)], c_v)

                def comb_row(r, _):
                    for j in range(feat // LANES):
                        sl = pl.ds(j * LANES, LANES)
                        drain_v[r, sl] = drain_v[r, sl] * alpha + c_v[r, sl] * beta
                    return 0
                lax.fori_loop(0, DC, comb_row, 0)
            pltpu.sync_copy(drain_v, out_ref.at[pl.ds(obase + k * DC, DC)])

    return spmm


def _tc_combine(ts, n, weight, prelu_a):
    b, _, fin = ts[0].shape
    k_order = weight.shape[0]
    fout = weight.shape[2]
    bn = 1000
    assert n % bn == 0

    def body(*refs):
        t_refs = refs[:k_order]
        w_ref, a_ref, o_ref = refs[k_order:]
        acc = jnp.zeros((bn, fout), jnp.float32)
        for k in range(k_order):
            acc = acc + jnp.dot(t_refs[k][0], w_ref[k],
                                preferred_element_type=jnp.float32)
        a = a_ref[0]
        o_ref[0] = jnp.where(acc >= 0, acc, a * acc)

    t_spec = pl.BlockSpec((1, bn, fin), lambda i, j: (i, j, 0))
    return pl.pallas_call(
        body,
        grid=(b, n // bn),
        in_specs=[t_spec] * k_order + [
            pl.BlockSpec((k_order, fin, fout), lambda i, j: (0, 0, 0)),
            pl.BlockSpec(memory_space=pltpu.SMEM),
        ],
        out_specs=pl.BlockSpec((1, bn, fout), lambda i, j: (i, j, 0)),
        out_shape=jax.ShapeDtypeStruct((b, n, fout), jnp.float32),
    )(*ts, weight, prelu_a)


def kernel(x, lap_indices, lap_values, weight, prelu_a):
    b, n, fin = x.shape
    k_order = weight.shape[0]
    e = lap_values.shape[0]

    src = lap_indices[0]
    dst = lap_indices[1]
    vals = lap_values
    pad = (-e) % (NUM_SUBCORES * CH * GC)
    if pad:
        src = jnp.concatenate([src, jnp.zeros((pad,), jnp.int32)])
        dst = jnp.concatenate([dst, jnp.zeros((pad,), jnp.int32)])
        vals = jnp.concatenate([vals, jnp.zeros((pad,), jnp.float32)])
    n_chunks_total = (e + pad) // CH
    val2d = vals.reshape(n_chunks_total, CH)

    n_pad = ((n + NUM_SUBCORES * DC - 1) // (NUM_SUBCORES * DC)) * (NUM_SUBCORES * DC)
    xp = jnp.pad(x, ((0, 0), (0, n_pad - n), (0, 0)))
    xf = xp.reshape(b * n_pad, fin)
    spmm_first = _make_spmm(n_pad, fin, n_chunks_total, 1.0, 0.0)
    spmm_rec = _make_spmm(n_pad, fin, n_chunks_total, 2.0, -1.0)

    ts = [xf, spmm_first(xf, xf, src, dst, val2d)]
    for _ in range(2, k_order):
        ts.append(spmm_rec(ts[-1], ts[-2], src, dst, val2d))

    ts3d = [t.reshape(b, n_pad, fin) for t in ts]
    return _tc_combine(ts3d, n, weight, prelu_a)


# speedup vs baseline: 25.6188x; 1.1319x over previous
import functools

import jax
import jax.numpy as jnp
from jax import lax
from jax.experimental import pallas as pl
from jax.experimental.pallas import tpu as pltpu
from jax.experimental.pallas import tpu_sc as plsc

NUM_CORES = 2
NUM_SUBCORES = 16
CH = 128
GC = 16
DC = 80
LANES = 16


def _make_spmm(n_nodes, feat, n_chunks_total, alpha, beta):
    assert n_nodes % (NUM_SUBCORES * DC) == 0
    assert n_chunks_total % (NUM_SUBCORES * GC) == 0
    npc = n_chunks_total // NUM_SUBCORES
    ngr = npc // GC
    rpt = n_nodes // NUM_SUBCORES
    n_drain = rpt // DC

    mesh = plsc.VectorSubcoreMesh(
        core_axis_name="c", subcore_axis_name="s",
        num_cores=NUM_CORES, num_subcores=NUM_SUBCORES)

    @functools.partial(
        pl.kernel,
        out_type=jax.ShapeDtypeStruct((NUM_CORES * n_nodes, feat), jnp.float32),
        mesh=mesh,
        scratch_types=[
            pltpu.VMEM_SHARED((n_nodes, feat), jnp.float32),
            pltpu.VMEM((GC, CH), jnp.float32),
            pltpu.VMEM((GC, 2, CH), jnp.int32),
            pltpu.VMEM((CH, feat), jnp.float32),
            pltpu.VMEM((DC, feat), jnp.float32),
            pltpu.VMEM((DC, feat), jnp.float32),
            pltpu.SemaphoreType.DMA,
        ],
    )
    def spmm(t_ref, c_ref, pair_ref, val_ref, out_ref,
             acc, val_v, pair_v, rows_v, drain_v, c_v, sem):
        cid = lax.axis_index("c")
        sid = lax.axis_index("s")
        cbase = sid * npc
        rbase = sid * rpt

        off = cid * n_nodes

        zeros = jnp.zeros((LANES,), jnp.float32)

        def zero_row(r, _):
            for j in range(feat // LANES):
                drain_v[r, pl.ds(j * LANES, LANES)] = zeros
            return 0
        lax.fori_loop(0, DC, zero_row, 0)
        for k in range(n_drain):
            pltpu.sync_copy(drain_v, acc.at[pl.ds(rbase + k * DC, DC)])
        plsc.subcore_barrier()

        def group_body(gi, _):
            gb = cbase + gi * GC
            pltpu.sync_copy(val_ref.at[pl.ds(gb, GC)], val_v)
            pltpu.sync_copy(pair_ref.at[pl.ds(gb, GC)], pair_v)

            def shift_row(i, _):
                for g in range(CH // LANES):
                    sl = pl.ds(g * LANES, LANES)
                    pair_v[i, 0, sl] = pair_v[i, 0, sl] + off
                return 0
            lax.fori_loop(0, GC, shift_row, 0)

            def chunk_body(i, _):
                pltpu.async_copy(t_ref.at[pair_v.at[i, 0]], rows_v, sem).wait()

                def scale_group(g, _):
                    gs = pl.multiple_of(g * LANES, LANES)
                    v16 = val_v[i, pl.ds(gs, LANES)]
                    for ei in range(LANES):
                        v = v16[ei]
                        e = gs + ei
                        for j in range(feat // LANES):
                            sl = pl.ds(j * LANES, LANES)
                            rows_v[e, sl] = rows_v[e, sl] * v
                    return 0
                lax.fori_loop(0, CH // LANES, scale_group, 0)
                pltpu.sync_copy(rows_v, acc.at[pair_v.at[i, 1]], add=True)
                return 0
            lax.fori_loop(0, GC, chunk_body, 0)
            return 0
        lax.fori_loop(0, ngr, group_body, 0)
        plsc.subcore_barrier()

        obase = cid * n_nodes + rbase
        for k in range(n_drain):
            pltpu.sync_copy(acc.at[pl.ds(rbase + k * DC, DC)], drain_v)
            if beta != 0.0:
                pltpu.sync_copy(c_ref.at[pl.ds(obase + k * DC, DC)], c_v)

                def comb_row(r, _):
                    for j in range(feat // LANES):
                        sl = pl.ds(j * LANES, LANES)
                        drain_v[r, sl] = drain_v[r, sl] * alpha + c_v[r, sl] * beta
                    return 0
                lax.fori_loop(0, DC, comb_row, 0)
            pltpu.sync_copy(drain_v, out_ref.at[pl.ds(obase + k * DC, DC)])

    return spmm


def _tc_combine(ts, n, weight, prelu_a):
    b, _, fin = ts[0].shape
    k_order = weight.shape[0]
    fout = weight.shape[2]
    bn = 1000
    assert n % bn == 0

    def body(*refs):
        t_refs = refs[:k_order]
        w_ref, a_ref, o_ref = refs[k_order:]
        acc = jnp.zeros((bn, fout), jnp.float32)
        for k in range(k_order):
            acc = acc + jnp.dot(t_refs[k][0], w_ref[k],
                                preferred_element_type=jnp.float32)
        a = a_ref[0]
        o_ref[0] = jnp.where(acc >= 0, acc, a * acc)

    t_spec = pl.BlockSpec((1, bn, fin), lambda i, j: (i, j, 0))
    return pl.pallas_call(
        body,
        grid=(b, n // bn),
        in_specs=[t_spec] * k_order + [
            pl.BlockSpec((k_order, fin, fout), lambda i, j: (0, 0, 0)),
            pl.BlockSpec(memory_space=pltpu.SMEM),
        ],
        out_specs=pl.BlockSpec((1, bn, fout), lambda i, j: (i, j, 0)),
        out_shape=jax.ShapeDtypeStruct((b, n, fout), jnp.float32),
    )(*ts, weight, prelu_a)


def kernel(x, lap_indices, lap_values, weight, prelu_a):
    b, n, fin = x.shape
    k_order = weight.shape[0]
    e = lap_values.shape[0]

    src = lap_indices[0]
    dst = lap_indices[1]
    vals = lap_values
    pad = (-e) % (NUM_SUBCORES * CH * GC)
    if pad:
        src = jnp.concatenate([src, jnp.zeros((pad,), jnp.int32)])
        dst = jnp.concatenate([dst, jnp.zeros((pad,), jnp.int32)])
        vals = jnp.concatenate([vals, jnp.zeros((pad,), jnp.float32)])
    n_chunks_total = (e + pad) // CH
    val2d = vals.reshape(n_chunks_total, CH)
    pair3d = jnp.stack([src.reshape(n_chunks_total, CH),
                        dst.reshape(n_chunks_total, CH)], axis=1)

    n_pad = ((n + NUM_SUBCORES * DC - 1) // (NUM_SUBCORES * DC)) * (NUM_SUBCORES * DC)
    xp = jnp.pad(x, ((0, 0), (0, n_pad - n), (0, 0)))
    xf = xp.reshape(b * n_pad, fin)
    spmm_first = _make_spmm(n_pad, fin, n_chunks_total, 1.0, 0.0)
    spmm_rec = _make_spmm(n_pad, fin, n_chunks_total, 2.0, -1.0)

    ts = [xf, spmm_first(xf, xf, pair3d, val2d)]
    for _ in range(2, k_order):
        ts.append(spmm_rec(ts[-1], ts[-2], pair3d, val2d))

    ts3d = [t.reshape(b, n_pad, fin) for t in ts]
    return _tc_combine(ts3d, n, weight, prelu_a)
